# baseline (speedup 1.0000x reference)
import functools

import jax
import jax.numpy as jnp
import numpy as np
from jax import lax
from jax.experimental import pallas as pl
from jax.experimental.pallas import tpu as pltpu
from jax.experimental.pallas import tpu_sc as plsc

N = 10000
E = 320000
HID = 128
B = 128
Q = 0.7
NPAD = 10240
STRIPE = NPAD // 16
K = 128
CH = 2560
EPAD = CH * K
NC, NS = 2, 16
NW = NC * NS
CPT_W = CH // NS
CPT_U = CH // NW

_PERM_MAT = np.zeros((B, B), np.float32)
_PERM_MAT[np.arange(B), np.random.default_rng(0).permutation(B)] = 1.0


def _zero_acc(zeros_hbm, acc_sh, sid):
    pltpu.sync_copy(zeros_hbm.at[pl.ds(sid * STRIPE, STRIPE)],
                    acc_sh.at[pl.ds(sid * STRIPE, STRIPE)])


def _sc_histogram_body(col_hbm, zeros1_hbm, out_hbm, col_v, ones_v, acc_sh):
    cid = lax.axis_index("c")
    sid = lax.axis_index("s")
    wid = sid * NC + cid
    _zero_acc(zeros1_hbm, acc_sh, sid)
    for g in range(K // 16):
        ones_v[pl.ds(g * 16, 16)] = jnp.full((16,), 1.0, jnp.float32)
    plsc.subcore_barrier()

    def body(i, carry):
        c = wid + i * NW
        pltpu.sync_copy(col_hbm.at[pl.ds(c * K, K)], col_v)
        pltpu.sync_copy(ones_v, acc_sh.at[col_v], add=True)
        return carry

    lax.fori_loop(0, CPT_U, body, 0)
    plsc.subcore_barrier()
    pltpu.sync_copy(acc_sh.at[pl.ds(sid * STRIPE, STRIPE)],
                    out_hbm.at[cid, pl.ds(sid * STRIPE, STRIPE)])


def _sc_agg_unweighted_body(table_hbm, row_hbm, col_hbm, zeros2_hbm, out_hbm,
                            row_v, col_v, rows_v, acc_sh, sg0):
    cid = lax.axis_index("c")
    sid = lax.axis_index("s")
    wid = sid * NC + cid
    _zero_acc(zeros2_hbm, acc_sh, sid)
    plsc.subcore_barrier()

    def body(i, carry):
        c = wid + i * NW
        pltpu.sync_copy(row_hbm.at[pl.ds(c * K, K)], row_v)
        pltpu.sync_copy(col_hbm.at[pl.ds(c * K, K)], col_v)
        pltpu.async_copy(table_hbm.at[row_v], rows_v, sg0).wait()
        pltpu.sync_copy(rows_v, acc_sh.at[col_v], add=True)
        return carry

    lax.fori_loop(0, CPT_U, body, 0)
    plsc.subcore_barrier()
    pltpu.sync_copy(acc_sh.at[pl.ds(sid * STRIPE, STRIPE)],
                    out_hbm.at[cid, pl.ds(sid * STRIPE, STRIPE)])


def _sc_edge_scores_body(s1_hbm, s2_hbm, row_hbm, col_hbm, zeros1_hbm,
                         ewc_hbm, ewb_hbm, deg_hbm,
                         row_v, col_v, g1_v, g2_v, ewc_v, ewb_v, acc_sh,
                         sa0, sa1):
    cid = lax.axis_index("c")
    sid = lax.axis_index("s")
    wid = sid * NC + cid
    _zero_acc(zeros1_hbm, acc_sh, sid)
    plsc.subcore_barrier()

    def body(i, carry):
        c0 = wid + i * NW
        pltpu.sync_copy(row_hbm.at[pl.ds(c0 * K, K)], row_v)
        pltpu.sync_copy(col_hbm.at[pl.ds(c0 * K, K)], col_v)
        pltpu.async_copy(s1_hbm.at[row_v], g1_v, sa0).wait()
        pltpu.async_copy(s2_hbm.at[col_v], g2_v, sa1).wait()
        for g in range(K // 16):
            sl = pl.ds(g * 16, 16)
            t = g1_v[sl] + g2_v[sl]
            ew = 1.0 / (1.0 + jnp.exp(-t))
            ewc_v[sl] = ew
            ewb_v[sl] = 1.0 - ew
        pltpu.sync_copy(ewc_v, ewc_hbm.at[pl.ds(c0 * K, K)])
        pltpu.sync_copy(ewb_v, ewb_hbm.at[pl.ds(c0 * K, K)])
        pltpu.sync_copy(ewc_v, acc_sh.at[col_v], add=True)
        return carry

    lax.fori_loop(0, CPT_U, body, 0)
    plsc.subcore_barrier()
    pltpu.sync_copy(acc_sh.at[pl.ds(sid * STRIPE, STRIPE)],
                    deg_hbm.at[cid, pl.ds(sid * STRIPE, STRIPE)])


def _sc_agg_weighted_body(tab2_hbm, ew2_hbm, row_hbm, col_hbm, zeros2_hbm,
                          out_hbm, row_v, col_v, ew_v, rows_v, acc_sh, sg0):
    cid = lax.axis_index("c")
    sid = lax.axis_index("s")
    _zero_acc(zeros2_hbm, acc_sh, sid)
    plsc.subcore_barrier()
    roff = jnp.zeros((16,), jnp.int32) + cid * NPAD

    def body(i, carry):
        c0 = sid + i * NS
        pltpu.sync_copy(row_hbm.at[pl.ds(c0 * K, K)], row_v)
        pltpu.sync_copy(col_hbm.at[pl.ds(c0 * K, K)], col_v)
        for g in range(K // 16):
            sl = pl.ds(g * 16, 16)
            row_v[sl] = row_v[sl] + roff
        pltpu.sync_copy(ew2_hbm.at[pl.ds(cid * EPAD + c0 * K, K)],
                        ew_v.at[pl.ds(0, K)])
        pltpu.async_copy(tab2_hbm.at[row_v], rows_v, sg0).wait()

        def edge(e, carry2):
            w = ew_v[pl.ds(e, 16)][0]
            for j in range(HID // 16):
                sl = pl.ds(j * 16, 16)
                rows_v[e, sl] = rows_v[e, sl] * w
            return carry2

        lax.fori_loop(0, K, edge, 0)
        pltpu.sync_copy(rows_v, acc_sh.at[col_v], add=True)
        return carry

    lax.fori_loop(0, CPT_W, body, 0)
    plsc.subcore_barrier()
    pltpu.sync_copy(acc_sh.at[pl.ds(sid * STRIPE, STRIPE)],
                    out_hbm.at[cid, pl.ds(sid * STRIPE, STRIPE)])


@functools.cache
def _sc_kernels():
    mesh = plsc.VectorSubcoreMesh(core_axis_name="c", subcore_axis_name="s")
    f32, i32 = jnp.float32, jnp.int32
    DMA = pltpu.SemaphoreType.DMA
    hist = pl.kernel(
        _sc_histogram_body, mesh=mesh,
        out_type=jax.ShapeDtypeStruct((2, NPAD), f32),
        scratch_types=[
            pltpu.VMEM((K,), i32),
            pltpu.VMEM((K,), f32),
            pltpu.VMEM_SHARED((NPAD,), f32),
        ])
    agg_u = pl.kernel(
        _sc_agg_unweighted_body, mesh=mesh,
        out_type=jax.ShapeDtypeStruct((2, NPAD, HID), f32),
        scratch_types=[
            pltpu.VMEM((K,), i32),
            pltpu.VMEM((K,), i32),
            pltpu.VMEM((K, HID), f32),
            pltpu.VMEM_SHARED((NPAD, HID), f32),
            DMA,
        ])
    edges = pl.kernel(
        _sc_edge_scores_body, mesh=mesh,
        out_type=(
            jax.ShapeDtypeStruct((EPAD,), f32),
            jax.ShapeDtypeStruct((EPAD,), f32),
            jax.ShapeDtypeStruct((2, NPAD), f32),
        ),
        scratch_types=[
            pltpu.VMEM((K,), i32),
            pltpu.VMEM((K,), i32),
            pltpu.VMEM((K,), f32),
            pltpu.VMEM((K,), f32),
            pltpu.VMEM((K,), f32),
            pltpu.VMEM((K,), f32),
            pltpu.VMEM_SHARED((NPAD,), f32),
            DMA, DMA,
        ])
    agg_w = pl.kernel(
        _sc_agg_weighted_body, mesh=mesh,
        out_type=jax.ShapeDtypeStruct((2, NPAD, HID), f32),
        scratch_types=[
            pltpu.VMEM((K,), i32),
            pltpu.VMEM((K,), i32),
            pltpu.VMEM((K + 16,), f32),
            pltpu.VMEM((K, HID), f32),
            pltpu.VMEM_SHARED((NPAD, HID), f32),
            DMA,
        ])
    return hist, agg_u, edges, agg_w


def _tc_call(fn, out_shapes, *args):
    flat_outs = [jax.ShapeDtypeStruct(s, d) for (s, d) in out_shapes]
    return pl.pallas_call(fn, out_shape=flat_outs)(*args)


def _rsqrt_deg(deg):
    return jnp.where(deg > 0, lax.rsqrt(jnp.maximum(deg, 1e-12)), 0.0)


_ZTAIL = ((NPAD - N, HID), jnp.float32)


def _ztail(ref, val):
    ref[:N] = val
    ref[N:] = jnp.zeros((NPAD - N,) + ref.shape[1:], jnp.float32)


def _tc_stage_a(x, cnt0, cnt1, m_fc_w, m_fc_b, m_c1_w):
    def body(x_r, c0_r, c1_r, w_r, b_r, w1_r, u1_o, dinv_o, cnt_o):
        cnt = c0_r[:N] + c1_r[:N]
        deg0 = cnt + 1.0
        dinv = _rsqrt_deg(deg0)
        h = jnp.dot(x_r[...], w_r[...].T, preferred_element_type=jnp.float32) + b_r[...]
        t1 = jnp.dot(h, w1_r[...].T, preferred_element_type=jnp.float32)
        _ztail(u1_o, dinv * t1)
        dinv_o[...] = dinv
        cnt_o[...] = cnt

    return _tc_call(body, [((NPAD, HID), jnp.float32), ((N, 1), jnp.float32),
                           ((N, 1), jnp.float32)],
                    x, cnt0, cnt1, m_fc_w, m_fc_b, m_c1_w)


def _tc_conv_next(a0, a1, u, dinv, bias, w_next):
    def body(a0_r, a1_r, u_r, d_r, b_r, w_r, un_o):
        agg = a0_r[:N] + a1_r[:N]
        h = jax.nn.relu(d_r[...] * (agg + u_r[:N]) + b_r[...])
        _ztail(un_o, d_r[...] * jnp.dot(h, w_r[...].T,
                                        preferred_element_type=jnp.float32))

    return _tc_call(body, [((NPAD, HID), jnp.float32)],
                    a0, a1, u, dinv, bias, w_next)[0]


def _tc_stage_c(a0, a1, u2, dinv0, m_c2_b, ns_wT, ns_b, es_w1T, es_w2T, es_b,
                x, gc_fc_w, gc_fc_b, gb_fc_w, gb_fc_b, gc_c1_w, gb_c1_w):
    def body(a0_r, a1_r, u2_r, d_r, b2_r, nsw_r, nsb_r, esw1_r, esw2_r, esb_r,
             x_r, gcw_r, gcb_r, gbw_r, gbb_r, gc1_r, gb1_r,
             s1_o, s2_o, tc1_o, tb1_o):
        agg = a0_r[:N] + a1_r[:N]
        h2 = jax.nn.relu(d_r[...] * (agg + u2_r[:N]) + b2_r[...])
        ns = jax.nn.sigmoid(
            jnp.dot(h2, nsw_r[...], preferred_element_type=jnp.float32)
            + nsb_r[...])
        zt1 = jnp.zeros((NPAD - N, 1), jnp.float32)
        s1_o[:N] = jnp.dot(h2, esw1_r[...],
                           preferred_element_type=jnp.float32) + esb_r[...]
        s1_o[N:] = zt1
        s2_o[:N] = jnp.dot(h2, esw2_r[...], preferred_element_type=jnp.float32)
        s2_o[N:] = zt1
        hc = jnp.dot(x_r[...] * ns, gcw_r[...].T,
                     preferred_element_type=jnp.float32) + gcb_r[...]
        hb = jnp.dot(x_r[...] * (1.0 - ns), gbw_r[...].T,
                     preferred_element_type=jnp.float32) + gbb_r[...]
        tc1_o[...] = jnp.dot(hc, gc1_r[...].T, preferred_element_type=jnp.float32)
        tb1_o[...] = jnp.dot(hb, gb1_r[...].T, preferred_element_type=jnp.float32)

    return _tc_call(body,
                    [((NPAD, 1), jnp.float32), ((NPAD, 1), jnp.float32),
                     ((N, HID), jnp.float32), ((N, HID), jnp.float32)],
                    a0, a1, u2, dinv0, m_c2_b, ns_wT, ns_b, es_w1T, es_w2T, es_b,
                    x, gc_fc_w, gc_fc_b, gb_fc_w, gb_fc_b, gc_c1_w, gb_c1_w)


def _tc_stage_d(d0, d1, cnt, tc1, tb1):
    def body(d0_r, d1_r, cnt_r, tc1_r, tb1_r, uc_o, ub_o, dc_o, db_o):
        deg_c = d0_r[:N] + d1_r[:N] + 1.0
        deg_b = cnt_r[...] + 2.0 - deg_c
        dc = _rsqrt_deg(deg_c)
        db = _rsqrt_deg(deg_b)
        _ztail(uc_o, dc * tc1_r[...])
        _ztail(ub_o, db * tb1_r[...])
        dc_o[...] = dc
        db_o[...] = db

    return _tc_call(body,
                    [((NPAD, HID), jnp.float32), ((NPAD, HID), jnp.float32),
                     ((N, 1), jnp.float32), ((N, 1), jnp.float32)],
                    d0, d1, cnt, tc1, tb1)


def _tc_conv_next_w(aggc, aggb, uc, ub, dc, db, bc, bb, wc_next, wb_next):
    def body(ac_r, ab_r, uc_r, ub_r, dc_r, db_r, bc_r, bb_r, wc_r, wb_r,
             unc_o, unb_o):
        hc = jax.nn.relu(dc_r[...] * (ac_r[:N] + uc_r[:N]) + bc_r[...])
        hb = jax.nn.relu(db_r[...] * (ab_r[:N] + ub_r[:N]) + bb_r[...])
        _ztail(unc_o, dc_r[...] * jnp.dot(hc, wc_r[...].T,
                                          preferred_element_type=jnp.float32))
        _ztail(unb_o, db_r[...] * jnp.dot(hb, wb_r[...].T,
                                          preferred_element_type=jnp.float32))

    return _tc_call(body, [((NPAD, HID), jnp.float32), ((NPAD, HID), jnp.float32)],
                    aggc, aggb, uc, ub, dc, db, bc, bb, wc_next, wb_next)


def _tc_stage_f(aggc, aggb, uc, ub, dc, db, bc, bb, batf, yf,
                wsplits, bscal, perm_mat):
    def body(ac_r, ab_r, uc_r, ub_r, dc_r, db_r, bc_r, bb_r, bat_r, y_r,
             wc0c, wc0b, wc1c, wc1b, wb0c, wb0b, wb1c, wb1b,
             bc0, bc1, bb0, bb1, pm_r, p0_o, p1_o, loss_o):
        hc = jax.nn.relu(dc_r[...] * (ac_r[:N] + uc_r[:N]) + bc_r[...])
        hb = jax.nn.relu(db_r[...] * (ab_r[:N] + ub_r[:N]) + bb_r[...])
        gid = lax.broadcasted_iota(jnp.int32, (B, N), 0).astype(jnp.float32)
        mask = (gid == bat_r[...]).astype(jnp.float32)
        cntb = jnp.sum(mask, axis=1, keepdims=True)
        inv = 1.0 / jnp.maximum(cntb, 1.0)
        score_c = jnp.dot(mask, hc, preferred_element_type=jnp.float32) * inv
        score_b = jnp.dot(mask, hb, preferred_element_type=jnp.float32) * inv

        zb_swap = jnp.dot(pm_r[...], score_b, preferred_element_type=jnp.float32)
        y_ = y_r[...]
        y_sw = jnp.dot(pm_r[...], y_, preferred_element_type=jnp.float32)

        def mm(a, w):
            return jnp.dot(a, w[...], preferred_element_type=jnp.float32)

        def heads(sb):
            pc0 = mm(score_c, wc0c) + mm(sb, wc0b) + bc0[...]
            pc1 = mm(score_c, wc1c) + mm(sb, wc1b) + bc1[...]
            pb0 = mm(score_c, wb0c) + mm(sb, wb0b) + bb0[...]
            pb1 = mm(score_c, wb1c) + mm(sb, wb1b) + bb1[...]
            return pc0, pc1, pb0, pb1

        def ce(l0, l1, t):
            m = jnp.maximum(l0, l1)
            lse = m + jnp.log(jnp.exp(l0 - m) + jnp.exp(l1 - m))
            pick = jnp.where(t == 0.0, l0, l1)
            return lse - pick

        def gce(l0, l1, t):
            c = ce(l0, l1, t)
            yg = jnp.exp(-c)
            lw = jnp.exp(Q * jnp.log(yg)) * Q
            return c * lw

        pc0, pc1, pb0, pb1 = heads(score_b)
        loss_vec = ce(pc0, pc1, y_) + gce(pb0, pb1, y_)
        mc0, mc1, ma0, ma1 = heads(zb_swap)
        loss_vec = loss_vec + 15.0 * (ce(mc0, mc1, y_) + gce(ma0, ma1, y_sw))
        p0_o[...] = pc0
        p1_o[...] = pc1
        loss_o[...] = jnp.sum(loss_vec * (1.0 / B), axis=0, keepdims=True)

    return _tc_call(body,
                    [((B, 1), jnp.float32), ((B, 1), jnp.float32),
                     ((1, 1), jnp.float32)],
                    aggc, aggb, uc, ub, dc, db, bc, bb, batf, yf,
                    *wsplits, *bscal, perm_mat)


def kernel(x, edge_index, batch, y,
           m_fc_w, m_fc_b, m_c1_w, m_c1_b, m_c2_w, m_c2_b,
           m_ns_w, m_ns_b, m_es_w, m_es_b,
           gc_fc_w, gc_fc_b, gc_c1_w, gc_c1_b, gc_c2_w, gc_c2_b,
           gb_fc_w, gb_fc_b, gb_c1_w, gb_c1_b, gb_c2_w, gb_c2_b,
           mc_w, mc_b, mb_w, mb_b):
    f32 = jnp.float32
    i32 = jnp.int32
    pad = jnp.full((EPAD - E,), N, i32)
    row2d = jnp.concatenate([edge_index[0], pad])
    col2d = jnp.concatenate([edge_index[1], pad])
    zeros1 = jnp.zeros((NPAD,), f32)
    zeros2 = jnp.zeros((NPAD, HID), f32)
    _sc_histogram, _sc_agg_unweighted, _sc_edge_scores, _sc_agg_weighted = _sc_kernels()

    cnt2 = _sc_histogram(col2d, zeros1)
    u1, dinv0, cnt = _tc_stage_a(x, cnt2[0][:, None], cnt2[1][:, None],
                                 m_fc_w, m_fc_b[None, :], m_c1_w)

    agg1 = _sc_agg_unweighted(u1, row2d, col2d, zeros2)
    u2 = _tc_conv_next(agg1[0], agg1[1], u1, dinv0, m_c1_b[None, :], m_c2_w)

    agg2 = _sc_agg_unweighted(u2, row2d, col2d, zeros2)
    s1, s2, tc1, tb1 = _tc_stage_c(agg2[0], agg2[1], u2, dinv0,
                                   m_c2_b[None, :],
                                   m_ns_w.T, m_ns_b[None, :],
                                   m_es_w[0, :HID][:, None],
                                   m_es_w[0, HID:][:, None],
                                   m_es_b[None, :],
                                   x, gc_fc_w, gc_fc_b[None, :],
                                   gb_fc_w, gb_fc_b[None, :],
                                   gc_c1_w, gb_c1_w)

    ewc, ewb, degc2 = _sc_edge_scores(s1.reshape(NPAD), s2.reshape(NPAD),
                                      row2d, col2d, zeros1)
    uc1, ub1, dc, db = _tc_stage_d(degc2[0][:, None], degc2[1][:, None],
                                   cnt, tc1, tb1)

    ew2 = jnp.concatenate([ewc, ewb])
    a1 = _sc_agg_weighted(jnp.concatenate([uc1, ub1]), ew2, row2d, col2d,
                          zeros2)
    uc2, ub2 = _tc_conv_next_w(a1[0], a1[1], uc1, ub1, dc, db,
                               gc_c1_b[None, :], gb_c1_b[None, :],
                               gc_c2_w, gb_c2_w)

    a2 = _sc_agg_weighted(jnp.concatenate([uc2, ub2]), ew2, row2d, col2d,
                          zeros2)
    ac2, ab2 = a2[0], a2[1]
    wsplits = [mc_w[0, :HID][:, None], mc_w[0, HID:][:, None],
               mc_w[1, :HID][:, None], mc_w[1, HID:][:, None],
               mb_w[0, :HID][:, None], mb_w[0, HID:][:, None],
               mb_w[1, :HID][:, None], mb_w[1, HID:][:, None]]
    bscal = [mc_b[0].reshape(1, 1), mc_b[1].reshape(1, 1),
             mb_b[0].reshape(1, 1), mb_b[1].reshape(1, 1)]
    p0, p1, loss = _tc_stage_f(ac2, ab2, uc2, ub2, dc, db,
                               gc_c2_b[None, :], gb_c2_b[None, :],
                               batch[None, :].astype(f32),
                               y[:, None].astype(f32),
                               wsplits, bscal, jnp.asarray(_PERM_MAT))

    pred_c = jnp.concatenate([p0, p1], axis=1)
    return pred_c, loss.reshape(()), y


# speedup vs baseline: 1.4718x; 1.4718x over previous
import functools

import jax
import jax.numpy as jnp
import numpy as np
from jax import lax
from jax.experimental import pallas as pl
from jax.experimental.pallas import tpu as pltpu
from jax.experimental.pallas import tpu_sc as plsc

N = 10000
E = 320000
HID = 128
B = 128
Q = 0.7
NPAD = 10240
STRIPE = NPAD // 16
K = 128
NCHUNK = E // K
NC, NS = 2, 16
NW = NC * NS

_PERM_MAT = np.zeros((B, B), np.float32)
_PERM_MAT[np.arange(B), np.random.default_rng(0).permutation(B)] = 1.0


def _zero_acc(zeros_hbm, acc_sh, sid):
    pltpu.sync_copy(zeros_hbm.at[pl.ds(sid * STRIPE, STRIPE)],
                    acc_sh.at[pl.ds(sid * STRIPE, STRIPE)])


def _sc_histogram_body(col_hbm, zeros1_hbm, out_hbm, col_v, ones_v, acc_sh, sem):
    cid = lax.axis_index("c")
    sid = lax.axis_index("s")
    wid = sid * NC + cid
    _zero_acc(zeros1_hbm, acc_sh, sid)
    for g in range(K // 16):
        ones_v[pl.ds(g * 16, 16)] = jnp.full((16,), 1.0, jnp.float32)
    plsc.subcore_barrier()

    def body(i, carry):
        c = wid + i * NW

        @pl.when(c < NCHUNK)
        def _():
            pltpu.sync_copy(col_hbm.at[pl.ds(c * K, K)], col_v)
            pltpu.sync_copy(ones_v, acc_sh.at[col_v], add=True)
        return carry

    lax.fori_loop(0, (NCHUNK + NW - 1) // NW, body, 0)
    plsc.subcore_barrier()
    pltpu.sync_copy(acc_sh.at[pl.ds(sid * STRIPE, STRIPE)],
                    out_hbm.at[cid, pl.ds(sid * STRIPE, STRIPE)])


def _sc_agg_unweighted_body(table_hbm, row_hbm, col_hbm, zeros2_hbm, out_hbm,
                            row_v, col_v, rows_v, acc_sh, sem):
    cid = lax.axis_index("c")
    sid = lax.axis_index("s")
    wid = sid * NC + cid
    _zero_acc(zeros2_hbm, acc_sh, sid)
    plsc.subcore_barrier()

    def body(i, carry):
        c = wid + i * NW

        @pl.when(c < NCHUNK)
        def _():
            pltpu.sync_copy(row_hbm.at[pl.ds(c * K, K)], row_v)
            pltpu.sync_copy(col_hbm.at[pl.ds(c * K, K)], col_v)
            pltpu.async_copy(table_hbm.at[row_v], rows_v, sem).wait()
            pltpu.sync_copy(rows_v, acc_sh.at[col_v], add=True)
        return carry

    lax.fori_loop(0, (NCHUNK + NW - 1) // NW, body, 0)
    plsc.subcore_barrier()
    pltpu.sync_copy(acc_sh.at[pl.ds(sid * STRIPE, STRIPE)],
                    out_hbm.at[cid, pl.ds(sid * STRIPE, STRIPE)])


def _sc_edge_scores_body(s1_hbm, s2_hbm, row_hbm, col_hbm, zeros1_hbm,
                         ewc_hbm, ewb_hbm, deg_hbm,
                         row_v, col_v, g1_v, g2_v, ewc_v, ewb_v, acc_sh, sem):
    cid = lax.axis_index("c")
    sid = lax.axis_index("s")
    wid = sid * NC + cid
    _zero_acc(zeros1_hbm, acc_sh, sid)
    plsc.subcore_barrier()

    def body(i, carry):
        c = wid + i * NW

        @pl.when(c < NCHUNK)
        def _():
            pltpu.sync_copy(row_hbm.at[pl.ds(c * K, K)], row_v)
            pltpu.sync_copy(col_hbm.at[pl.ds(c * K, K)], col_v)
            pltpu.async_copy(s1_hbm.at[row_v], g1_v, sem).wait()
            pltpu.async_copy(s2_hbm.at[col_v], g2_v, sem).wait()
            for g in range(K // 16):
                sl = pl.ds(g * 16, 16)
                t = g1_v[sl] + g2_v[sl]
                ew = 1.0 / (1.0 + jnp.exp(-t))
                ewc_v[sl] = ew
                ewb_v[sl] = 1.0 - ew
            pltpu.sync_copy(ewc_v, ewc_hbm.at[pl.ds(c * K, K)])
            pltpu.sync_copy(ewb_v, ewb_hbm.at[pl.ds(c * K, K)])
            pltpu.sync_copy(ewc_v, acc_sh.at[col_v], add=True)
        return carry

    lax.fori_loop(0, (NCHUNK + NW - 1) // NW, body, 0)
    plsc.subcore_barrier()
    pltpu.sync_copy(acc_sh.at[pl.ds(sid * STRIPE, STRIPE)],
                    deg_hbm.at[cid, pl.ds(sid * STRIPE, STRIPE)])


def _sc_agg_weighted_body(tabc_hbm, tabb_hbm, ewc_hbm, ewb_hbm, row_hbm, col_hbm,
                          zeros2_hbm, outc_hbm, outb_hbm,
                          row_v, col_v, ew_v, rows_v, acc_sh, sem):
    cid = lax.axis_index("c")
    sid = lax.axis_index("s")
    _zero_acc(zeros2_hbm, acc_sh, sid)
    plsc.subcore_barrier()

    def chunk(c):
        pltpu.sync_copy(row_hbm.at[pl.ds(c * K, K)], row_v)
        pltpu.sync_copy(col_hbm.at[pl.ds(c * K, K)], col_v)

        @pl.when(cid == 0)
        def _():
            pltpu.sync_copy(ewc_hbm.at[pl.ds(c * K, K)], ew_v.at[pl.ds(0, K)])
            pltpu.async_copy(tabc_hbm.at[row_v], rows_v, sem).wait()

        @pl.when(cid == 1)
        def _():
            pltpu.sync_copy(ewb_hbm.at[pl.ds(c * K, K)], ew_v.at[pl.ds(0, K)])
            pltpu.async_copy(tabb_hbm.at[row_v], rows_v, sem).wait()

        def edge(e, carry):
            w = ew_v[pl.ds(e, 16)][0]
            for j in range(HID // 16):
                sl = pl.ds(j * 16, 16)
                rows_v[e, sl] = rows_v[e, sl] * w
            return carry

        lax.fori_loop(0, K, edge, 0)
        pltpu.sync_copy(rows_v, acc_sh.at[col_v], add=True)

    def body(i, carry):
        c = sid + i * NS

        @pl.when(c < NCHUNK)
        def _():
            chunk(c)
        return carry

    lax.fori_loop(0, (NCHUNK + NS - 1) // NS, body, 0)
    plsc.subcore_barrier()
    sl = pl.ds(sid * STRIPE, STRIPE)

    @pl.when(cid == 0)
    def _():
        pltpu.sync_copy(acc_sh.at[sl], outc_hbm.at[sl])

    @pl.when(cid == 1)
    def _():
        pltpu.sync_copy(acc_sh.at[sl], outb_hbm.at[sl])


@functools.cache
def _sc_kernels():
    mesh = plsc.VectorSubcoreMesh(core_axis_name="c", subcore_axis_name="s")
    f32, i32 = jnp.float32, jnp.int32
    hist = pl.kernel(
        _sc_histogram_body, mesh=mesh,
        out_type=jax.ShapeDtypeStruct((2, NPAD), f32),
        scratch_types=[
            pltpu.VMEM((K,), i32),
            pltpu.VMEM((K,), f32),
            pltpu.VMEM_SHARED((NPAD,), f32),
            pltpu.SemaphoreType.DMA,
        ])
    agg_u = pl.kernel(
        _sc_agg_unweighted_body, mesh=mesh,
        out_type=jax.ShapeDtypeStruct((2, NPAD, HID), f32),
        scratch_types=[
            pltpu.VMEM((K,), i32),
            pltpu.VMEM((K,), i32),
            pltpu.VMEM((K, HID), f32),
            pltpu.VMEM_SHARED((NPAD, HID), f32),
            pltpu.SemaphoreType.DMA,
        ])
    edges = pl.kernel(
        _sc_edge_scores_body, mesh=mesh,
        out_type=(
            jax.ShapeDtypeStruct((E,), f32),
            jax.ShapeDtypeStruct((E,), f32),
            jax.ShapeDtypeStruct((2, NPAD), f32),
        ),
        scratch_types=[
            pltpu.VMEM((K,), i32),
            pltpu.VMEM((K,), i32),
            pltpu.VMEM((K,), f32),
            pltpu.VMEM((K,), f32),
            pltpu.VMEM((K,), f32),
            pltpu.VMEM((K,), f32),
            pltpu.VMEM_SHARED((NPAD,), f32),
            pltpu.SemaphoreType.DMA,
        ])
    agg_w = pl.kernel(
        _sc_agg_weighted_body, mesh=mesh,
        out_type=(
            jax.ShapeDtypeStruct((NPAD, HID), f32),
            jax.ShapeDtypeStruct((NPAD, HID), f32),
        ),
        scratch_types=[
            pltpu.VMEM((K,), i32),
            pltpu.VMEM((K,), i32),
            pltpu.VMEM((K + 16,), f32),
            pltpu.VMEM((K, HID), f32),
            pltpu.VMEM_SHARED((NPAD, HID), f32),
            pltpu.SemaphoreType.DMA,
        ])
    return hist, agg_u, edges, agg_w


def _tc_call(fn, out_shapes, *args):
    flat_outs = [jax.ShapeDtypeStruct(s, d) for (s, d) in out_shapes]
    return pl.pallas_call(fn, out_shape=flat_outs)(*args)


def _rsqrt_deg(deg):
    return jnp.where(deg > 0, lax.rsqrt(jnp.maximum(deg, 1e-12)), 0.0)


def _tc_stage_a(x, cnt0, cnt1, m_fc_w, m_fc_b, m_c1_w):
    def body(x_r, c0_r, c1_r, w_r, b_r, w1_r, u1_o, dinv_o, cnt_o):
        cnt = c0_r[:N] + c1_r[:N]
        deg0 = cnt + 1.0
        dinv = _rsqrt_deg(deg0)
        h = jnp.dot(x_r[...], w_r[...].T, preferred_element_type=jnp.float32) + b_r[...]
        t1 = jnp.dot(h, w1_r[...].T, preferred_element_type=jnp.float32)
        u1_o[...] = dinv * t1
        dinv_o[...] = dinv
        cnt_o[...] = cnt

    return _tc_call(body, [((N, HID), jnp.float32), ((N, 1), jnp.float32),
                           ((N, 1), jnp.float32)],
                    x, cnt0, cnt1, m_fc_w, m_fc_b, m_c1_w)


def _tc_conv_next(a0, a1, u, dinv, bias, w_next):
    def body(a0_r, a1_r, u_r, d_r, b_r, w_r, un_o, h_o):
        agg = a0_r[:N] + a1_r[:N]
        h = jax.nn.relu(d_r[...] * (agg + u_r[...]) + b_r[...])
        un_o[...] = d_r[...] * jnp.dot(h, w_r[...].T, preferred_element_type=jnp.float32)
        h_o[...] = h

    return _tc_call(body, [((N, HID), jnp.float32), ((N, HID), jnp.float32)],
                    a0, a1, u, dinv, bias, w_next)


def _tc_stage_c(a0, a1, u2, dinv0, m_c2_b, ns_wT, ns_b, es_w1T, es_w2T, es_b,
                x, gc_fc_w, gc_fc_b, gb_fc_w, gb_fc_b, gc_c1_w, gb_c1_w):
    def body(a0_r, a1_r, u2_r, d_r, b2_r, nsw_r, nsb_r, esw1_r, esw2_r, esb_r,
             x_r, gcw_r, gcb_r, gbw_r, gbb_r, gc1_r, gb1_r,
             s1_o, s2_o, tc1_o, tb1_o):
        agg = a0_r[:N] + a1_r[:N]
        h2 = jax.nn.relu(d_r[...] * (agg + u2_r[...]) + b2_r[...])
        ns = jax.nn.sigmoid(
            jnp.dot(h2, nsw_r[...], preferred_element_type=jnp.float32)
            + nsb_r[...])
        s1_o[...] = jnp.dot(h2, esw1_r[...],
                            preferred_element_type=jnp.float32) + esb_r[...]
        s2_o[...] = jnp.dot(h2, esw2_r[...], preferred_element_type=jnp.float32)
        hc = jnp.dot(x_r[...] * ns, gcw_r[...].T,
                     preferred_element_type=jnp.float32) + gcb_r[...]
        hb = jnp.dot(x_r[...] * (1.0 - ns), gbw_r[...].T,
                     preferred_element_type=jnp.float32) + gbb_r[...]
        tc1_o[...] = jnp.dot(hc, gc1_r[...].T, preferred_element_type=jnp.float32)
        tb1_o[...] = jnp.dot(hb, gb1_r[...].T, preferred_element_type=jnp.float32)

    return _tc_call(body,
                    [((N, 1), jnp.float32), ((N, 1), jnp.float32),
                     ((N, HID), jnp.float32), ((N, HID), jnp.float32)],
                    a0, a1, u2, dinv0, m_c2_b, ns_wT, ns_b, es_w1T, es_w2T, es_b,
                    x, gc_fc_w, gc_fc_b, gb_fc_w, gb_fc_b, gc_c1_w, gb_c1_w)


def _tc_stage_d(d0, d1, cnt, tc1, tb1):
    def body(d0_r, d1_r, cnt_r, tc1_r, tb1_r, uc_o, ub_o, dc_o, db_o):
        deg_c = d0_r[:N] + d1_r[:N] + 1.0
        deg_b = cnt_r[...] + 2.0 - deg_c
        dc = _rsqrt_deg(deg_c)
        db = _rsqrt_deg(deg_b)
        uc_o[...] = dc * tc1_r[...]
        ub_o[...] = db * tb1_r[...]
        dc_o[...] = dc
        db_o[...] = db

    return _tc_call(body,
                    [((N, HID), jnp.float32), ((N, HID), jnp.float32),
                     ((N, 1), jnp.float32), ((N, 1), jnp.float32)],
                    d0, d1, cnt, tc1, tb1)


def _tc_conv_next_w(aggc, aggb, uc, ub, dc, db, bc, bb, wc_next, wb_next):
    def body(ac_r, ab_r, uc_r, ub_r, dc_r, db_r, bc_r, bb_r, wc_r, wb_r,
             unc_o, unb_o):
        hc = jax.nn.relu(dc_r[...] * (ac_r[:N] + uc_r[...]) + bc_r[...])
        hb = jax.nn.relu(db_r[...] * (ab_r[:N] + ub_r[...]) + bb_r[...])
        unc_o[...] = dc_r[...] * jnp.dot(hc, wc_r[...].T, preferred_element_type=jnp.float32)
        unb_o[...] = db_r[...] * jnp.dot(hb, wb_r[...].T, preferred_element_type=jnp.float32)

    return _tc_call(body, [((N, HID), jnp.float32), ((N, HID), jnp.float32)],
                    aggc, aggb, uc, ub, dc, db, bc, bb, wc_next, wb_next)


def _tc_stage_f(aggc, aggb, uc, ub, dc, db, bc, bb, batf, yf,
                wsplits, bscal, perm_mat):
    def body(ac_r, ab_r, uc_r, ub_r, dc_r, db_r, bc_r, bb_r, bat_r, y_r,
             wc0c, wc0b, wc1c, wc1b, wb0c, wb0b, wb1c, wb1b,
             bc0, bc1, bb0, bb1, pm_r, p0_o, p1_o, loss_o):
        hc = jax.nn.relu(dc_r[...] * (ac_r[:N] + uc_r[...]) + bc_r[...])
        hb = jax.nn.relu(db_r[...] * (ab_r[:N] + ub_r[...]) + bb_r[...])
        gid = lax.broadcasted_iota(jnp.int32, (B, N), 0).astype(jnp.float32)
        mask = (gid == bat_r[...]).astype(jnp.float32)
        cntb = jnp.sum(mask, axis=1, keepdims=True)
        inv = 1.0 / jnp.maximum(cntb, 1.0)
        score_c = jnp.dot(mask, hc, preferred_element_type=jnp.float32) * inv
        score_b = jnp.dot(mask, hb, preferred_element_type=jnp.float32) * inv

        zb_swap = jnp.dot(pm_r[...], score_b, preferred_element_type=jnp.float32)
        y_ = y_r[...]
        y_sw = jnp.dot(pm_r[...], y_, preferred_element_type=jnp.float32)

        def mm(a, w):
            return jnp.dot(a, w[...], preferred_element_type=jnp.float32)

        def heads(sb):
            pc0 = mm(score_c, wc0c) + mm(sb, wc0b) + bc0[...]
            pc1 = mm(score_c, wc1c) + mm(sb, wc1b) + bc1[...]
            pb0 = mm(score_c, wb0c) + mm(sb, wb0b) + bb0[...]
            pb1 = mm(score_c, wb1c) + mm(sb, wb1b) + bb1[...]
            return pc0, pc1, pb0, pb1

        def ce(l0, l1, t):
            m = jnp.maximum(l0, l1)
            lse = m + jnp.log(jnp.exp(l0 - m) + jnp.exp(l1 - m))
            pick = jnp.where(t == 0.0, l0, l1)
            return lse - pick

        def gce(l0, l1, t):
            c = ce(l0, l1, t)
            yg = jnp.exp(-c)
            lw = jnp.exp(Q * jnp.log(yg)) * Q
            return c * lw

        pc0, pc1, pb0, pb1 = heads(score_b)
        loss_vec = ce(pc0, pc1, y_) + gce(pb0, pb1, y_)
        mc0, mc1, ma0, ma1 = heads(zb_swap)
        loss_vec = loss_vec + 15.0 * (ce(mc0, mc1, y_) + gce(ma0, ma1, y_sw))
        p0_o[...] = pc0
        p1_o[...] = pc1
        loss_o[...] = jnp.sum(loss_vec * (1.0 / B), axis=0, keepdims=True)

    return _tc_call(body,
                    [((B, 1), jnp.float32), ((B, 1), jnp.float32),
                     ((1, 1), jnp.float32)],
                    aggc, aggb, uc, ub, dc, db, bc, bb, batf, yf,
                    *wsplits, *bscal, perm_mat)


def kernel(x, edge_index, batch, y,
           m_fc_w, m_fc_b, m_c1_w, m_c1_b, m_c2_w, m_c2_b,
           m_ns_w, m_ns_b, m_es_w, m_es_b,
           gc_fc_w, gc_fc_b, gc_c1_w, gc_c1_b, gc_c2_w, gc_c2_b,
           gb_fc_w, gb_fc_b, gb_c1_w, gb_c1_b, gb_c2_w, gb_c2_b,
           mc_w, mc_b, mb_w, mb_b):
    f32 = jnp.float32
    row = edge_index[0]
    col = edge_index[1]
    zeros1 = jnp.zeros((NPAD,), f32)
    zeros2 = jnp.zeros((NPAD, HID), f32)
    _sc_histogram, _sc_agg_unweighted, _sc_edge_scores, _sc_agg_weighted = _sc_kernels()

    cnt2 = _sc_histogram(col, zeros1)
    u1, dinv0, cnt = _tc_stage_a(x, cnt2[0][:, None], cnt2[1][:, None],
                                 m_fc_w, m_fc_b[None, :], m_c1_w)

    agg1 = _sc_agg_unweighted(u1, row, col, zeros2)
    u2, _h1 = _tc_conv_next(agg1[0], agg1[1], u1, dinv0, m_c1_b[None, :],
                            m_c2_w)

    agg2 = _sc_agg_unweighted(u2, row, col, zeros2)
    s1, s2, tc1, tb1 = _tc_stage_c(agg2[0], agg2[1], u2, dinv0,
                                   m_c2_b[None, :],
                                   m_ns_w.T, m_ns_b[None, :],
                                   m_es_w[0, :HID][:, None],
                                   m_es_w[0, HID:][:, None],
                                   m_es_b[None, :],
                                   x, gc_fc_w, gc_fc_b[None, :],
                                   gb_fc_w, gb_fc_b[None, :],
                                   gc_c1_w, gb_c1_w)

    ewc, ewb, degc2 = _sc_edge_scores(s1.reshape(N), s2.reshape(N), row, col,
                                      zeros1)
    uc1, ub1, dc, db = _tc_stage_d(degc2[0][:, None], degc2[1][:, None],
                                   cnt, tc1, tb1)

    ac1, ab1 = _sc_agg_weighted(uc1, ub1, ewc, ewb, row, col, zeros2)
    uc2, ub2 = _tc_conv_next_w(ac1, ab1, uc1, ub1, dc, db,
                               gc_c1_b[None, :], gb_c1_b[None, :],
                               gc_c2_w, gb_c2_w)

    ac2, ab2 = _sc_agg_weighted(uc2, ub2, ewc, ewb, row, col, zeros2)
    wsplits = [mc_w[0, :HID][:, None], mc_w[0, HID:][:, None],
               mc_w[1, :HID][:, None], mc_w[1, HID:][:, None],
               mb_w[0, :HID][:, None], mb_w[0, HID:][:, None],
               mb_w[1, :HID][:, None], mb_w[1, HID:][:, None]]
    bscal = [mc_b[0].reshape(1, 1), mc_b[1].reshape(1, 1),
             mb_b[0].reshape(1, 1), mb_b[1].reshape(1, 1)]
    p0, p1, loss = _tc_stage_f(ac2, ab2, uc2, ub2, dc, db,
                               gc_c2_b[None, :], gb_c2_b[None, :],
                               batch[None, :].astype(f32),
                               y[:, None].astype(f32),
                               wsplits, bscal, jnp.asarray(_PERM_MAT))

    pred_c = jnp.concatenate([p0, p1], axis=1)
    return pred_c, loss.reshape(()), y
